# Initial kernel scaffold; baseline (speedup 1.0000x reference)
import jax
import jax.numpy as jnp
from jax.experimental import pallas as pl


def kernel(x, edge_index, conv_feat, Wc1, bc1, Wc2, bc2, W1, as1, ad1, b1, W2, as2, ad2, b2, W3, as3, ad3, b3, W4, as4, ad4, b4, W5, as5, ad5, b5):
    raise NotImplementedError("write your pallas kernel here")


# speedup vs baseline: 19.2307x; 19.2307x over previous
import functools

import jax
import jax.numpy as jnp
from jax import lax
from jax.experimental import pallas as pl
from jax.experimental.pallas import tpu as pltpu
from jax.experimental.pallas import tpu_sc as plsc

N = 10000
NT = 10008
E = 160000
EP = 172032
ROWS = 400
GRID = 25
NSUB = 16
NCORE = 2
PER_SUB = EP // (NSUB * NCORE)
CHUNKS = PER_SUB // 128


def _proj_kernel(h_ref, w_ref, o_ref):
    o_ref[...] = jnp.dot(h_ref[...], w_ref[...], preferred_element_type=jnp.float32)


def _att_kernel(h_ref, ws_ref, wd_ref, as_ref, ad_ref, gm_ref):
    i = pl.program_id(0)
    h = h_ref[...]
    a_s = jnp.dot(h, ws_ref[...], preferred_element_type=jnp.float32)
    as_ref[...] = a_s
    ad_ref[...] = jnp.dot(h, wd_ref[...], preferred_element_type=jnp.float32)
    row = i * ROWS + lax.broadcasted_iota(jnp.int32, (ROWS, 16), 0)
    masked = jnp.where(row < N, a_s, -1e30)
    bmax = jnp.max(masked, axis=0, keepdims=True)
    bmax8 = jnp.broadcast_to(bmax, (8, 16))

    @pl.when(i == 0)
    def _():
        gm_ref[...] = jnp.full((8, 16), -1e30, jnp.float32)

    gm_ref[...] = jnp.maximum(gm_ref[...], bmax8)


def _selu(v):
    return jnp.where(v > 0, 1.0507009873554805 * v,
                     1.7580993408473766 * (jnp.exp(v) - 1.0))


def _fix01(x, c0, c1):
    upper = x[:, 0:1] == 1.0
    down = x[:, 0:1] == 0.0
    left = x[:, 1:2] == 0.0
    right = x[:, 1:2] == 1.0
    c0 = jnp.where(down, 0.0, jnp.where(upper, 1.0, c0))
    c1 = jnp.where(right, 1.0, jnp.where(left, 0.0, c1))
    return c0, c1


def _epi_kernel(x_ref, b_ref, *acc_refs, inv_h):
    o_ref = acc_refs[-1]
    accs = acc_refs[:-1]
    parts = [(accs[2 * i][...] + accs[2 * i + 1][...]) for i in range(len(accs) // 2)]
    y = jnp.concatenate(parts, axis=1) if len(parts) > 1 else parts[0]
    y = _selu(y * inv_h + b_ref[0:1, :])
    c0, c1 = _fix01(x_ref[...], y[:, 0:1], y[:, 1:2])
    o_ref[...] = jnp.concatenate([c0, c1, y[:, 2:]], axis=1)


def _epi5_kernel(x_ref, b_ref, a0_ref, a1_ref, o_ref):
    x = x_ref[...]
    a = a0_ref[...] + a1_ref[...]
    y0 = jnp.sum(a[:, 0:16], axis=1, keepdims=True)
    y1 = jnp.sum(a[:, 16:32], axis=1, keepdims=True)
    y = jnp.concatenate([y0, y1], axis=1) * (1.0 / 16.0) + b_ref[0:1, 0:2]
    y = x[:, 0:2] + _selu(y)
    c0, c1 = _fix01(x, y[:, 0:1], y[:, 1:2])
    o_ref[...] = jnp.concatenate([c0, c1], axis=1)


def _tc_proj(h_in, w):
    fin = h_in.shape[1]
    d = w.shape[1]
    return pl.pallas_call(
        _proj_kernel,
        grid=(GRID,),
        in_specs=[pl.BlockSpec((ROWS, fin), lambda i: (i, 0)),
                  pl.BlockSpec((fin, d), lambda i: (0, 0))],
        out_specs=pl.BlockSpec((ROWS, d), lambda i: (i, 0)),
        out_shape=jax.ShapeDtypeStruct((N, d), jnp.float32),
    )(h_in, w)


def _tc_att(h_in, ws, wd):
    fin = h_in.shape[1]
    return pl.pallas_call(
        _att_kernel,
        grid=(GRID,),
        in_specs=[pl.BlockSpec((ROWS, fin), lambda i: (i, 0)),
                  pl.BlockSpec((fin, 16), lambda i: (0, 0)),
                  pl.BlockSpec((fin, 16), lambda i: (0, 0))],
        out_specs=[pl.BlockSpec((ROWS, 16), lambda i: (i, 0)),
                   pl.BlockSpec((ROWS, 16), lambda i: (i, 0)),
                   pl.BlockSpec((8, 16), lambda i: (0, 0))],
        out_shape=[jax.ShapeDtypeStruct((N, 16), jnp.float32),
                   jax.ShapeDtypeStruct((N, 16), jnp.float32),
                   jax.ShapeDtypeStruct((8, 16), jnp.float32)],
    )(h_in, ws, wd)


def _tc_epi(x, bpad, acc_parts, inv_h, fout):
    n_parts = len(acc_parts) // 2
    cw = acc_parts[0].shape[1]
    specs = [pl.BlockSpec((ROWS, 10), lambda i: (i, 0)),
             pl.BlockSpec((8, cw * n_parts), lambda i: (0, 0))]
    for a in acc_parts:
        specs.append(pl.BlockSpec((ROWS, a.shape[1]), lambda i: (i, 0)))
    return pl.pallas_call(
        functools.partial(_epi_kernel, inv_h=inv_h),
        grid=(GRID,),
        in_specs=specs,
        out_specs=pl.BlockSpec((ROWS, fout), lambda i: (i, 0)),
        out_shape=jax.ShapeDtypeStruct((N, fout), jnp.float32),
    )(x, bpad, *acc_parts)


def _tc_epi5(x, bpad, a0, a1):
    return pl.pallas_call(
        _epi5_kernel,
        grid=(GRID,),
        in_specs=[pl.BlockSpec((ROWS, 10), lambda i: (i, 0)),
                  pl.BlockSpec((8, 16), lambda i: (0, 0)),
                  pl.BlockSpec((ROWS, 32), lambda i: (i, 0)),
                  pl.BlockSpec((ROWS, 32), lambda i: (i, 0))],
        out_specs=pl.BlockSpec((ROWS, 2), lambda i: (i, 0)),
        out_shape=jax.ShapeDtypeStruct((N, 2), jnp.float32),
    )(x, bpad, a0, a1)


_MESH = plsc.VectorSubcoreMesh(core_axis_name="c", subcore_axis_name="s")
_SC_PARAMS = pltpu.CompilerParams(use_tc_tiling_on_sc=False)


def _leaky(v):
    return jnp.where(v > 0.0, v, 0.2 * v)


def _pass1_body(src_hbm, dst_hbm, asrc_hbm, adst_hbm, gmax_hbm, zer_hbm,
                ee_hbm, s_hbm,
                idxs, idxd, arows, arowd, eebuf, gv, sacc, sem):
    cid = lax.axis_index("c")
    sid = lax.axis_index("s")

    @pl.when(sid == 0)
    def _():
        pltpu.sync_copy(zer_hbm, sacc)

    pltpu.sync_copy(gmax_hbm, gv)
    plsc.subcore_barrier()
    base0 = (cid * NSUB + sid) * PER_SUB
    gvec = gv[...]

    def chunk(k, _):
        base = base0 + k * 128
        pltpu.sync_copy(src_hbm.at[pl.ds(base, 128)], idxs)
        pltpu.sync_copy(dst_hbm.at[pl.ds(base, 128)], idxd)
        pltpu.async_copy(asrc_hbm.at[idxs], arows, sem).wait()
        pltpu.async_copy(adst_hbm.at[idxd], arowd, sem).wait()

        def inner(j, _):
            a_s = arows[j]
            a_d = arowd[j]
            ee = jnp.exp(_leaky(a_s + a_d) - _leaky(gvec + a_d))
            eebuf[j] = ee
            return 0

        lax.fori_loop(0, 128, inner, 0)
        pltpu.sync_copy(eebuf, ee_hbm.at[pl.ds(base, 128)])
        pltpu.sync_copy(eebuf, sacc.at[idxd], add=True)
        return 0

    lax.fori_loop(0, CHUNKS, chunk, 0)
    plsc.subcore_barrier()

    @pl.when(sid == 0)
    def _():
        pltpu.sync_copy(sacc, s_hbm.at[cid])


@functools.partial(
    pl.kernel, mesh=_MESH, compiler_params=_SC_PARAMS,
    out_type=[jax.ShapeDtypeStruct((EP, 16), jnp.float32),
              jax.ShapeDtypeStruct((2, NT, 16), jnp.float32)],
    scratch_types=[pltpu.VMEM((128,), jnp.int32),
                   pltpu.VMEM((128,), jnp.int32),
                   pltpu.VMEM((128, 16), jnp.float32),
                   pltpu.VMEM((128, 16), jnp.float32),
                   pltpu.VMEM((128, 16), jnp.float32),
                   pltpu.VMEM((16,), jnp.float32),
                   pltpu.VMEM_SHARED((NT, 16), jnp.float32),
                   pltpu.SemaphoreType.DMA],
)
def _sc_pass1(*args):
    _pass1_body(*args)


def _pass2_body(src_hbm, dst_hbm, ee_hbm, rr_hbm, h_hbm, zer_hbm,
                acc_hbm,
                idxs, idxd, eebuf, rrows, webuf, hh, msgbuf, sacc, sem,
                *, d_width, cw, heads, l5mode):
    cid = lax.axis_index("c")
    sid = lax.axis_index("s")

    @pl.when(sid == 0)
    def _():
        pltpu.sync_copy(zer_hbm, sacc)

    plsc.subcore_barrier()
    base0 = (cid * NSUB + sid) * PER_SUB

    def chunk(k, _):
        base = base0 + k * 128
        pltpu.sync_copy(src_hbm.at[pl.ds(base, 128)], idxs)
        pltpu.sync_copy(dst_hbm.at[pl.ds(base, 128)], idxd)
        pltpu.sync_copy(ee_hbm.at[pl.ds(base, 128)], eebuf)
        pltpu.async_copy(rr_hbm.at[idxd], rrows, sem).wait()

        def wmul(j, _):
            webuf[j] = eebuf[j] * rrows[j]
            return 0

        lax.fori_loop(0, 128, wmul, 0)
        sub = hh.shape[0]

        def subchunk(sj, _):
            pltpu.async_copy(h_hbm.at[idxs.at[pl.ds(sj * sub, sub)]], hh, sem).wait()

            def edge(r, _):
                row = sj * sub + r
                if l5mode:
                    wv = webuf[row]
                    msgbuf[row, 0:16] = wv * hh[r, 0:16]
                    msgbuf[row, 16:32] = wv * hh[r, 16:32]
                else:
                    wv = webuf[row]
                    gdn = lax.GatherDimensionNumbers(
                        offset_dims=(), collapsed_slice_dims=(0,),
                        start_index_map=(0,))
                    accs = [jnp.zeros((16,), jnp.float32) for _ in range(cw // 16)]
                    for h in range(heads):
                        h16 = jnp.full((16, 1), h, jnp.int32)
                        wb = lax.gather(wv, h16, gdn, (1,),
                                        mode=lax.GatherScatterMode.PROMISE_IN_BOUNDS)
                        for c in range(cw // 16):
                            accs[c] = accs[c] + wb * hh[r, h * cw + c * 16: h * cw + (c + 1) * 16]
                    for c in range(cw // 16):
                        msgbuf[row, c * 16:(c + 1) * 16] = accs[c]
                return 0

            lax.fori_loop(0, sub, edge, 0)
            return 0

        lax.fori_loop(0, 128 // sub, subchunk, 0)
        pltpu.sync_copy(msgbuf, sacc.at[idxd], add=True)
        return 0

    lax.fori_loop(0, CHUNKS, chunk, 0)
    plsc.subcore_barrier()

    @pl.when(sid == 0)
    def _():
        pltpu.sync_copy(sacc, acc_hbm.at[cid])


@functools.cache
def _make_pass2(d_width, cw, heads, l5mode):
    body = functools.partial(_pass2_body, d_width=d_width, cw=cw,
                             heads=heads, l5mode=l5mode)
    sub = min(32, max(8, 16384 // d_width))

    @functools.partial(
        pl.kernel, mesh=_MESH, compiler_params=_SC_PARAMS,
        out_type=[jax.ShapeDtypeStruct((2, NT, cw), jnp.float32)],
        scratch_types=[pltpu.VMEM((128,), jnp.int32),
                       pltpu.VMEM((128,), jnp.int32),
                       pltpu.VMEM((128, 16), jnp.float32),
                       pltpu.VMEM((128, 16), jnp.float32),
                       pltpu.VMEM((128, 16), jnp.float32),
                       pltpu.VMEM((sub, d_width), jnp.float32),
                       pltpu.VMEM((128, cw), jnp.float32),
                       pltpu.VMEM_SHARED((NT, cw), jnp.float32),
                       pltpu.SemaphoreType.DMA],
    )
    def _k(*args):
        body(*args)

    return _k


def _fold_att(w, att, heads, fout):
    ws = (w.reshape(-1, heads, fout) * att[None]).sum(-1)
    return jnp.pad(ws, ((0, 0), (0, 16 - heads)))


def _gat_sc(h_in, srcp, dstp, x, w_parts, ws, wd, bpad, heads, fout, l5mode):
    asrc, adst, gmax8 = _tc_att(h_in, ws, wd)
    asrc = jnp.pad(asrc, ((0, NT - N), (0, 0)))
    adst = jnp.pad(adst, ((0, NT - N), (0, 0)))
    gmax = gmax8[0]
    zer16 = jnp.zeros((NT, 16), jnp.float32)
    ee, s2 = _sc_pass1(srcp, dstp, asrc, adst, gmax, zer16)
    s = s2[0, :N] + s2[1, :N]
    rr = 1.0 / (s + 1e-16)
    rrp = jnp.pad(rr, ((0, NT - N), (0, 0)))
    acc_parts = []
    for wp in w_parts:
        htab = _tc_proj(h_in, wp)
        d_width = wp.shape[1]
        cw = 32 if l5mode else d_width // heads
        zcw = jnp.zeros((NT, cw), jnp.float32)
        (acc2,) = _make_pass2(d_width, cw, heads, l5mode)(
            srcp, dstp, ee, rrp, htab, zcw)
        acc_parts.append(acc2[0, :N])
        acc_parts.append(acc2[1, :N])
    return acc_parts


def kernel(x, edge_index, conv_feat, Wc1, bc1, Wc2, bc2, W1, as1, ad1, b1, W2, as2, ad2, b2, W3, as3, ad3, b3, W4, as4, ad4, b4, W5, as5, ad5, b5):
    n = x.shape[0]
    dn = ('NCHW', 'OIHW', 'NCHW')
    f = jax.lax.conv_general_dilated(conv_feat, Wc1, (1, 1), 'SAME', dimension_numbers=dn) + bc1[None, :, None, None]
    f = jax.nn.relu(f)
    f = jax.lax.conv_general_dilated(f, Wc2, (1, 1), 'SAME', dimension_numbers=dn) + bc2[None, :, None, None]
    f = jax.nn.relu(f)
    f = jnp.mean(f, axis=(2, 3))
    cf = jnp.broadcast_to(f, (n, 24))
    h = jnp.concatenate([cf, x], axis=1)

    loop = jnp.arange(n, dtype=jnp.int32)
    srcp = jnp.concatenate([edge_index[0].astype(jnp.int32), loop,
                            jnp.zeros((EP - E - n,), jnp.int32)])
    dstp = jnp.concatenate([edge_index[1].astype(jnp.int32), loop,
                            jnp.full((EP - E - n,), N, jnp.int32)])

    dims = [(34, 64, 8), (64, 128, 16), (128, 256, 8), (256, 128, 8)]
    weights = [(W1, as1, ad1, b1), (W2, as2, ad2, b2), (W3, as3, ad3, b3), (W4, as4, ad4, b4)]
    for (fin, fout, heads), (W, a_s, a_d, b) in zip(dims, weights):
        ws = _fold_att(W, a_s, heads, fout)
        wd = _fold_att(W, a_d, heads, fout)
        if fout == 256:
            wr = W.reshape(fin, heads, fout)
            w_parts = [wr[:, :, :128].reshape(fin, heads * 128),
                       wr[:, :, 128:].reshape(fin, heads * 128)]
        else:
            w_parts = [W]
        bpad = jnp.broadcast_to(b[None, :], (8, fout))
        acc_parts = _gat_sc(h, srcp, dstp, x, w_parts, ws, wd, bpad, heads, fout, False)
        h = _tc_epi(x, bpad, acc_parts, 1.0 / heads, fout)

    ws5 = _fold_att(W5, as5, 16, 2)
    wd5 = _fold_att(W5, ad5, 16, 2)
    w5fm = W5.reshape(128, 16, 2).transpose(0, 2, 1).reshape(128, 32)
    b5pad = jnp.broadcast_to(jnp.pad(b5, (0, 14))[None, :], (8, 16))
    acc_parts = _gat_sc(h, srcp, dstp, x, [w5fm], ws5, wd5, b5pad, 16, 2, True)
    out = _tc_epi5(x, b5pad, acc_parts[0], acc_parts[1])
    return out


# speedup vs baseline: 19.4255x; 1.0101x over previous
import functools

import jax
import jax.numpy as jnp
from jax import lax
from jax.experimental import pallas as pl
from jax.experimental.pallas import tpu as pltpu
from jax.experimental.pallas import tpu_sc as plsc

N = 10000
NT = 10008
E = 160000
EP = 172032
ROWS = 400
GRID = 25
NSUB = 16
NCORE = 2
PER_SUB = EP // (NSUB * NCORE)
CHUNKS = PER_SUB // 128


def _proj_kernel(h_ref, w_ref, o_ref):
    o_ref[...] = jnp.dot(h_ref[...], w_ref[...], preferred_element_type=jnp.float32)


def _att_kernel(h_ref, ws_ref, wd_ref, as_ref, ad_ref, gm_ref):
    i = pl.program_id(0)
    h = h_ref[...]
    a_s = jnp.dot(h, ws_ref[...], preferred_element_type=jnp.float32)
    as_ref[...] = a_s
    ad_ref[...] = jnp.dot(h, wd_ref[...], preferred_element_type=jnp.float32)
    row = i * ROWS + lax.broadcasted_iota(jnp.int32, (ROWS, 16), 0)
    masked = jnp.where(row < N, a_s, -1e30)
    bmax = jnp.max(masked, axis=0, keepdims=True)
    bmax8 = jnp.broadcast_to(bmax, (8, 16))

    @pl.when(i == 0)
    def _():
        gm_ref[...] = jnp.full((8, 16), -1e30, jnp.float32)

    gm_ref[...] = jnp.maximum(gm_ref[...], bmax8)


def _selu(v):
    return jnp.where(v > 0, 1.0507009873554805 * v,
                     1.7580993408473766 * (jnp.exp(v) - 1.0))


def _fix01(x, c0, c1):
    upper = x[:, 0:1] == 1.0
    down = x[:, 0:1] == 0.0
    left = x[:, 1:2] == 0.0
    right = x[:, 1:2] == 1.0
    c0 = jnp.where(down, 0.0, jnp.where(upper, 1.0, c0))
    c1 = jnp.where(right, 1.0, jnp.where(left, 0.0, c1))
    return c0, c1


def _epi_kernel(x_ref, b_ref, *acc_refs, inv_h):
    o_ref = acc_refs[-1]
    accs = acc_refs[:-1]
    parts = [(accs[2 * i][...] + accs[2 * i + 1][...]) for i in range(len(accs) // 2)]
    y = jnp.concatenate(parts, axis=1) if len(parts) > 1 else parts[0]
    y = _selu(y * inv_h + b_ref[0:1, :])
    c0, c1 = _fix01(x_ref[...], y[:, 0:1], y[:, 1:2])
    o_ref[...] = jnp.concatenate([c0, c1, y[:, 2:]], axis=1)


def _epi5_kernel(x_ref, b_ref, a0_ref, a1_ref, o_ref):
    x = x_ref[...]
    a = a0_ref[...] + a1_ref[...]
    y0 = jnp.sum(a[:, 0:16], axis=1, keepdims=True)
    y1 = jnp.sum(a[:, 16:32], axis=1, keepdims=True)
    y = jnp.concatenate([y0, y1], axis=1) * (1.0 / 16.0) + b_ref[0:1, 0:2]
    y = x[:, 0:2] + _selu(y)
    c0, c1 = _fix01(x, y[:, 0:1], y[:, 1:2])
    o_ref[...] = jnp.concatenate([c0, c1], axis=1)


def _tc_proj(h_in, w):
    fin = h_in.shape[1]
    d = w.shape[1]
    return pl.pallas_call(
        _proj_kernel,
        grid=(GRID,),
        in_specs=[pl.BlockSpec((ROWS, fin), lambda i: (i, 0)),
                  pl.BlockSpec((fin, d), lambda i: (0, 0))],
        out_specs=pl.BlockSpec((ROWS, d), lambda i: (i, 0)),
        out_shape=jax.ShapeDtypeStruct((N, d), jnp.float32),
    )(h_in, w)


def _tc_att(h_in, ws, wd):
    fin = h_in.shape[1]
    return pl.pallas_call(
        _att_kernel,
        grid=(GRID,),
        in_specs=[pl.BlockSpec((ROWS, fin), lambda i: (i, 0)),
                  pl.BlockSpec((fin, 16), lambda i: (0, 0)),
                  pl.BlockSpec((fin, 16), lambda i: (0, 0))],
        out_specs=[pl.BlockSpec((ROWS, 16), lambda i: (i, 0)),
                   pl.BlockSpec((ROWS, 16), lambda i: (i, 0)),
                   pl.BlockSpec((8, 16), lambda i: (0, 0))],
        out_shape=[jax.ShapeDtypeStruct((N, 16), jnp.float32),
                   jax.ShapeDtypeStruct((N, 16), jnp.float32),
                   jax.ShapeDtypeStruct((8, 16), jnp.float32)],
    )(h_in, ws, wd)


def _tc_epi(x, bpad, acc_parts, inv_h, fout):
    n_parts = len(acc_parts) // 2
    cw = acc_parts[0].shape[1]
    specs = [pl.BlockSpec((ROWS, 10), lambda i: (i, 0)),
             pl.BlockSpec((8, cw * n_parts), lambda i: (0, 0))]
    for a in acc_parts:
        specs.append(pl.BlockSpec((ROWS, a.shape[1]), lambda i: (i, 0)))
    return pl.pallas_call(
        functools.partial(_epi_kernel, inv_h=inv_h),
        grid=(GRID,),
        in_specs=specs,
        out_specs=pl.BlockSpec((ROWS, fout), lambda i: (i, 0)),
        out_shape=jax.ShapeDtypeStruct((N, fout), jnp.float32),
    )(x, bpad, *acc_parts)


def _tc_epi5(x, bpad, a0, a1):
    return pl.pallas_call(
        _epi5_kernel,
        grid=(GRID,),
        in_specs=[pl.BlockSpec((ROWS, 10), lambda i: (i, 0)),
                  pl.BlockSpec((8, 16), lambda i: (0, 0)),
                  pl.BlockSpec((ROWS, 32), lambda i: (i, 0)),
                  pl.BlockSpec((ROWS, 32), lambda i: (i, 0))],
        out_specs=pl.BlockSpec((ROWS, 2), lambda i: (i, 0)),
        out_shape=jax.ShapeDtypeStruct((N, 2), jnp.float32),
    )(x, bpad, a0, a1)


_MESH = plsc.VectorSubcoreMesh(core_axis_name="c", subcore_axis_name="s")
_SC_PARAMS = pltpu.CompilerParams(use_tc_tiling_on_sc=False)


def _leaky(v):
    return jnp.where(v > 0.0, v, 0.2 * v)


def _pass1_body(src_hbm, dst_hbm, asrc_hbm, adst_hbm, gmax_hbm, zer_hbm,
                ee_hbm, s_hbm,
                idxs, idxd, arows, arowd, eebuf, gv, sacc, sem):
    cid = lax.axis_index("c")
    sid = lax.axis_index("s")

    @pl.when(sid == 0)
    def _():
        pltpu.sync_copy(zer_hbm, sacc)

    pltpu.sync_copy(gmax_hbm, gv)
    plsc.subcore_barrier()
    base0 = (cid * NSUB + sid) * PER_SUB
    gvec = gv[...]

    def chunk(k, _):
        base = base0 + k * 128
        pltpu.sync_copy(src_hbm.at[pl.ds(base, 128)], idxs)
        pltpu.sync_copy(dst_hbm.at[pl.ds(base, 128)], idxd)
        pltpu.async_copy(asrc_hbm.at[idxs], arows, sem).wait()
        pltpu.async_copy(adst_hbm.at[idxd], arowd, sem).wait()

        @plsc.parallel_loop(0, 128, 1, unroll=8)
        def inner(j):
            a_s = arows[j]
            a_d = arowd[j]
            eebuf[j] = jnp.exp(_leaky(a_s + a_d) - _leaky(gvec + a_d))
        pltpu.sync_copy(eebuf, ee_hbm.at[pl.ds(base, 128)])
        pltpu.sync_copy(eebuf, sacc.at[idxd], add=True)
        return 0

    lax.fori_loop(0, CHUNKS, chunk, 0)
    plsc.subcore_barrier()

    @pl.when(sid == 0)
    def _():
        pltpu.sync_copy(sacc, s_hbm.at[cid])


@functools.partial(
    pl.kernel, mesh=_MESH, compiler_params=_SC_PARAMS,
    out_type=[jax.ShapeDtypeStruct((EP, 16), jnp.float32),
              jax.ShapeDtypeStruct((2, NT, 16), jnp.float32)],
    scratch_types=[pltpu.VMEM((128,), jnp.int32),
                   pltpu.VMEM((128,), jnp.int32),
                   pltpu.VMEM((128, 16), jnp.float32),
                   pltpu.VMEM((128, 16), jnp.float32),
                   pltpu.VMEM((128, 16), jnp.float32),
                   pltpu.VMEM((16,), jnp.float32),
                   pltpu.VMEM_SHARED((NT, 16), jnp.float32),
                   pltpu.SemaphoreType.DMA],
)
def _sc_pass1(*args):
    _pass1_body(*args)


def _pass2_body(src_hbm, dst_hbm, ee_hbm, rr_hbm, h_hbm, zer_hbm,
                acc_hbm,
                idxs, idxd, eebuf, rrows, webuf, hh, msgbuf, sacc, sem,
                *, d_width, cw, heads, l5mode):
    cid = lax.axis_index("c")
    sid = lax.axis_index("s")

    @pl.when(sid == 0)
    def _():
        pltpu.sync_copy(zer_hbm, sacc)

    plsc.subcore_barrier()
    base0 = (cid * NSUB + sid) * PER_SUB

    def chunk(k, _):
        base = base0 + k * 128
        pltpu.sync_copy(src_hbm.at[pl.ds(base, 128)], idxs)
        pltpu.sync_copy(dst_hbm.at[pl.ds(base, 128)], idxd)
        pltpu.sync_copy(ee_hbm.at[pl.ds(base, 128)], eebuf)
        pltpu.async_copy(rr_hbm.at[idxd], rrows, sem).wait()

        @plsc.parallel_loop(0, 128, 1, unroll=8)
        def wmul(j):
            webuf[j] = eebuf[j] * rrows[j]

        sub = hh.shape[0]

        def subchunk(sj, _):
            pltpu.async_copy(h_hbm.at[idxs.at[pl.ds(sj * sub, sub)]], hh, sem).wait()

            @plsc.parallel_loop(0, sub, 1, unroll=2)
            def edge(r):
                row = sj * sub + r
                if l5mode:
                    wv = webuf[row]
                    msgbuf[row, 0:16] = wv * hh[r, 0:16]
                    msgbuf[row, 16:32] = wv * hh[r, 16:32]
                else:
                    wv = webuf[row]
                    gdn = lax.GatherDimensionNumbers(
                        offset_dims=(), collapsed_slice_dims=(0,),
                        start_index_map=(0,))
                    accs = [jnp.zeros((16,), jnp.float32) for _ in range(cw // 16)]
                    for h in range(heads):
                        h16 = jnp.full((16, 1), h, jnp.int32)
                        wb = lax.gather(wv, h16, gdn, (1,),
                                        mode=lax.GatherScatterMode.PROMISE_IN_BOUNDS)
                        for c in range(cw // 16):
                            accs[c] = accs[c] + wb * hh[r, h * cw + c * 16: h * cw + (c + 1) * 16]
                    for c in range(cw // 16):
                        msgbuf[row, c * 16:(c + 1) * 16] = accs[c]

            return 0

        lax.fori_loop(0, 128 // sub, subchunk, 0)
        pltpu.sync_copy(msgbuf, sacc.at[idxd], add=True)
        return 0

    lax.fori_loop(0, CHUNKS, chunk, 0)
    plsc.subcore_barrier()

    @pl.when(sid == 0)
    def _():
        pltpu.sync_copy(sacc, acc_hbm.at[cid])


@functools.cache
def _make_pass2(d_width, cw, heads, l5mode):
    body = functools.partial(_pass2_body, d_width=d_width, cw=cw,
                             heads=heads, l5mode=l5mode)
    sub = min(32, max(8, 16384 // d_width))

    @functools.partial(
        pl.kernel, mesh=_MESH, compiler_params=_SC_PARAMS,
        out_type=[jax.ShapeDtypeStruct((2, NT, cw), jnp.float32)],
        scratch_types=[pltpu.VMEM((128,), jnp.int32),
                       pltpu.VMEM((128,), jnp.int32),
                       pltpu.VMEM((128, 16), jnp.float32),
                       pltpu.VMEM((128, 16), jnp.float32),
                       pltpu.VMEM((128, 16), jnp.float32),
                       pltpu.VMEM((sub, d_width), jnp.float32),
                       pltpu.VMEM((128, cw), jnp.float32),
                       pltpu.VMEM_SHARED((NT, cw), jnp.float32),
                       pltpu.SemaphoreType.DMA],
    )
    def _k(*args):
        body(*args)

    return _k


def _fold_att(w, att, heads, fout):
    ws = (w.reshape(-1, heads, fout) * att[None]).sum(-1)
    return jnp.pad(ws, ((0, 0), (0, 16 - heads)))


def _gat_sc(h_in, srcp, dstp, x, w_parts, ws, wd, bpad, heads, fout, l5mode):
    asrc, adst, gmax8 = _tc_att(h_in, ws, wd)
    asrc = jnp.pad(asrc, ((0, NT - N), (0, 0)))
    adst = jnp.pad(adst, ((0, NT - N), (0, 0)))
    gmax = gmax8[0]
    zer16 = jnp.zeros((NT, 16), jnp.float32)
    ee, s2 = _sc_pass1(srcp, dstp, asrc, adst, gmax, zer16)
    s = s2[0, :N] + s2[1, :N]
    rr = 1.0 / (s + 1e-16)
    rrp = jnp.pad(rr, ((0, NT - N), (0, 0)))
    acc_parts = []
    for wp in w_parts:
        htab = _tc_proj(h_in, wp)
        d_width = wp.shape[1]
        cw = 32 if l5mode else d_width // heads
        zcw = jnp.zeros((NT, cw), jnp.float32)
        (acc2,) = _make_pass2(d_width, cw, heads, l5mode)(
            srcp, dstp, ee, rrp, htab, zcw)
        acc_parts.append(acc2[0, :N])
        acc_parts.append(acc2[1, :N])
    return acc_parts


def kernel(x, edge_index, conv_feat, Wc1, bc1, Wc2, bc2, W1, as1, ad1, b1, W2, as2, ad2, b2, W3, as3, ad3, b3, W4, as4, ad4, b4, W5, as5, ad5, b5):
    n = x.shape[0]
    dn = ('NCHW', 'OIHW', 'NCHW')
    f = jax.lax.conv_general_dilated(conv_feat, Wc1, (1, 1), 'SAME', dimension_numbers=dn) + bc1[None, :, None, None]
    f = jax.nn.relu(f)
    f = jax.lax.conv_general_dilated(f, Wc2, (1, 1), 'SAME', dimension_numbers=dn) + bc2[None, :, None, None]
    f = jax.nn.relu(f)
    f = jnp.mean(f, axis=(2, 3))
    cf = jnp.broadcast_to(f, (n, 24))
    h = jnp.concatenate([cf, x], axis=1)

    loop = jnp.arange(n, dtype=jnp.int32)
    srcp = jnp.concatenate([edge_index[0].astype(jnp.int32), loop,
                            jnp.zeros((EP - E - n,), jnp.int32)])
    dstp = jnp.concatenate([edge_index[1].astype(jnp.int32), loop,
                            jnp.full((EP - E - n,), N, jnp.int32)])

    dims = [(34, 64, 8), (64, 128, 16), (128, 256, 8), (256, 128, 8)]
    weights = [(W1, as1, ad1, b1), (W2, as2, ad2, b2), (W3, as3, ad3, b3), (W4, as4, ad4, b4)]
    for (fin, fout, heads), (W, a_s, a_d, b) in zip(dims, weights):
        ws = _fold_att(W, a_s, heads, fout)
        wd = _fold_att(W, a_d, heads, fout)
        if fout == 256:
            wr = W.reshape(fin, heads, fout)
            w_parts = [wr[:, :, :128].reshape(fin, heads * 128),
                       wr[:, :, 128:].reshape(fin, heads * 128)]
        else:
            w_parts = [W]
        bpad = jnp.broadcast_to(b[None, :], (8, fout))
        acc_parts = _gat_sc(h, srcp, dstp, x, w_parts, ws, wd, bpad, heads, fout, False)
        h = _tc_epi(x, bpad, acc_parts, 1.0 / heads, fout)

    ws5 = _fold_att(W5, as5, 16, 2)
    wd5 = _fold_att(W5, ad5, 16, 2)
    w5fm = W5.reshape(128, 16, 2).transpose(0, 2, 1).reshape(128, 32)
    b5pad = jnp.broadcast_to(jnp.pad(b5, (0, 14))[None, :], (8, 16))
    acc_parts = _gat_sc(h, srcp, dstp, x, [w5fm], ws5, wd5, b5pad, 16, 2, True)
    out = _tc_epi5(x, b5pad, acc_parts[0], acc_parts[1])
    return out


# speedup vs baseline: 19.5412x; 1.0060x over previous
import functools

import jax
import jax.numpy as jnp
from jax import lax
from jax.experimental import pallas as pl
from jax.experimental.pallas import tpu as pltpu
from jax.experimental.pallas import tpu_sc as plsc

N = 10000
NT = 10008
E = 160000
EP = 172032
ROWS = 400
GRID = 25
NSUB = 16
NCORE = 2
PER_SUB = EP // (NSUB * NCORE)
CHUNKS = PER_SUB // 128
CHUNK2 = 64


def _proj_kernel(h_ref, w_ref, o_ref):
    o_ref[...] = jnp.dot(h_ref[...], w_ref[...], preferred_element_type=jnp.float32)


def _att_kernel(h_ref, ws_ref, wd_ref, as_ref, ad_ref, gm_ref):
    i = pl.program_id(0)
    h = h_ref[...]
    a_s = jnp.dot(h, ws_ref[...], preferred_element_type=jnp.float32)
    as_ref[...] = a_s
    ad_ref[...] = jnp.dot(h, wd_ref[...], preferred_element_type=jnp.float32)
    row = i * ROWS + lax.broadcasted_iota(jnp.int32, (ROWS, 16), 0)
    masked = jnp.where(row < N, a_s, -1e30)
    bmax = jnp.max(masked, axis=0, keepdims=True)
    bmax8 = jnp.broadcast_to(bmax, (8, 16))

    @pl.when(i == 0)
    def _():
        gm_ref[...] = jnp.full((8, 16), -1e30, jnp.float32)

    gm_ref[...] = jnp.maximum(gm_ref[...], bmax8)


def _selu(v):
    return jnp.where(v > 0, 1.0507009873554805 * v,
                     1.7580993408473766 * (jnp.exp(v) - 1.0))


def _fix01(x, c0, c1):
    upper = x[:, 0:1] == 1.0
    down = x[:, 0:1] == 0.0
    left = x[:, 1:2] == 0.0
    right = x[:, 1:2] == 1.0
    c0 = jnp.where(down, 0.0, jnp.where(upper, 1.0, c0))
    c1 = jnp.where(right, 1.0, jnp.where(left, 0.0, c1))
    return c0, c1


def _epi_kernel(x_ref, b_ref, *acc_refs, inv_h):
    o_ref = acc_refs[-1]
    accs = acc_refs[:-1]
    parts = [(accs[2 * i][...] + accs[2 * i + 1][...]) for i in range(len(accs) // 2)]
    y = jnp.concatenate(parts, axis=1) if len(parts) > 1 else parts[0]
    y = _selu(y * inv_h + b_ref[0:1, :])
    c0, c1 = _fix01(x_ref[...], y[:, 0:1], y[:, 1:2])
    o_ref[...] = jnp.concatenate([c0, c1, y[:, 2:]], axis=1)


def _epi5_kernel(x_ref, b_ref, a0_ref, a1_ref, o_ref):
    x = x_ref[...]
    a = a0_ref[...] + a1_ref[...]
    y0 = jnp.sum(a[:, 0:16], axis=1, keepdims=True)
    y1 = jnp.sum(a[:, 16:32], axis=1, keepdims=True)
    y = jnp.concatenate([y0, y1], axis=1) * (1.0 / 16.0) + b_ref[0:1, 0:2]
    y = x[:, 0:2] + _selu(y)
    c0, c1 = _fix01(x, y[:, 0:1], y[:, 1:2])
    o_ref[...] = jnp.concatenate([c0, c1], axis=1)


def _tc_proj(h_in, w):
    fin = h_in.shape[1]
    d = w.shape[1]
    return pl.pallas_call(
        _proj_kernel,
        grid=(GRID,),
        in_specs=[pl.BlockSpec((ROWS, fin), lambda i: (i, 0)),
                  pl.BlockSpec((fin, d), lambda i: (0, 0))],
        out_specs=pl.BlockSpec((ROWS, d), lambda i: (i, 0)),
        out_shape=jax.ShapeDtypeStruct((N, d), jnp.float32),
    )(h_in, w)


def _tc_att(h_in, ws, wd):
    fin = h_in.shape[1]
    return pl.pallas_call(
        _att_kernel,
        grid=(GRID,),
        in_specs=[pl.BlockSpec((ROWS, fin), lambda i: (i, 0)),
                  pl.BlockSpec((fin, 16), lambda i: (0, 0)),
                  pl.BlockSpec((fin, 16), lambda i: (0, 0))],
        out_specs=[pl.BlockSpec((ROWS, 16), lambda i: (i, 0)),
                   pl.BlockSpec((ROWS, 16), lambda i: (i, 0)),
                   pl.BlockSpec((8, 16), lambda i: (0, 0))],
        out_shape=[jax.ShapeDtypeStruct((N, 16), jnp.float32),
                   jax.ShapeDtypeStruct((N, 16), jnp.float32),
                   jax.ShapeDtypeStruct((8, 16), jnp.float32)],
    )(h_in, ws, wd)


def _tc_epi(x, bpad, acc_parts, inv_h, fout):
    n_parts = len(acc_parts) // 2
    cw = acc_parts[0].shape[1]
    specs = [pl.BlockSpec((ROWS, 10), lambda i: (i, 0)),
             pl.BlockSpec((8, cw * n_parts), lambda i: (0, 0))]
    for a in acc_parts:
        specs.append(pl.BlockSpec((ROWS, a.shape[1]), lambda i: (i, 0)))
    return pl.pallas_call(
        functools.partial(_epi_kernel, inv_h=inv_h),
        grid=(GRID,),
        in_specs=specs,
        out_specs=pl.BlockSpec((ROWS, fout), lambda i: (i, 0)),
        out_shape=jax.ShapeDtypeStruct((N, fout), jnp.float32),
    )(x, bpad, *acc_parts)


def _tc_epi5(x, bpad, a0, a1):
    return pl.pallas_call(
        _epi5_kernel,
        grid=(GRID,),
        in_specs=[pl.BlockSpec((ROWS, 10), lambda i: (i, 0)),
                  pl.BlockSpec((8, 16), lambda i: (0, 0)),
                  pl.BlockSpec((ROWS, 32), lambda i: (i, 0)),
                  pl.BlockSpec((ROWS, 32), lambda i: (i, 0))],
        out_specs=pl.BlockSpec((ROWS, 2), lambda i: (i, 0)),
        out_shape=jax.ShapeDtypeStruct((N, 2), jnp.float32),
    )(x, bpad, a0, a1)


_MESH = plsc.VectorSubcoreMesh(core_axis_name="c", subcore_axis_name="s")
_SC_PARAMS = pltpu.CompilerParams(use_tc_tiling_on_sc=False)


def _leaky(v):
    return jnp.where(v > 0.0, v, 0.2 * v)


def _pass1_body(src_hbm, dst_hbm, asrc_hbm, adst_hbm, gmax_hbm, zer_hbm,
                ee_hbm, s_hbm,
                idxs, idxd, arows, arowd, eebuf, gv, sacc, sem):
    cid = lax.axis_index("c")
    sid = lax.axis_index("s")

    @pl.when(sid == 0)
    def _():
        pltpu.sync_copy(zer_hbm, sacc)

    pltpu.sync_copy(gmax_hbm, gv)
    plsc.subcore_barrier()
    base0 = (cid * NSUB + sid) * PER_SUB
    gvec = gv[...]

    def chunk(k, _):
        base = base0 + k * 128
        pltpu.sync_copy(src_hbm.at[pl.ds(base, 128)], idxs)
        pltpu.sync_copy(dst_hbm.at[pl.ds(base, 128)], idxd)
        pltpu.async_copy(asrc_hbm.at[idxs], arows, sem).wait()
        pltpu.async_copy(adst_hbm.at[idxd], arowd, sem).wait()

        @plsc.parallel_loop(0, 128, 1, unroll=8)
        def inner(j):
            a_s = arows[j]
            a_d = arowd[j]
            eebuf[j] = jnp.exp(_leaky(a_s + a_d) - _leaky(gvec + a_d))
        pltpu.sync_copy(eebuf, ee_hbm.at[pl.ds(base, 128)])
        pltpu.sync_copy(eebuf, sacc.at[idxd], add=True)
        return 0

    lax.fori_loop(0, CHUNKS, chunk, 0)
    plsc.subcore_barrier()

    @pl.when(sid == 0)
    def _():
        pltpu.sync_copy(sacc, s_hbm.at[cid])


@functools.partial(
    pl.kernel, mesh=_MESH, compiler_params=_SC_PARAMS,
    out_type=[jax.ShapeDtypeStruct((EP, 16), jnp.float32),
              jax.ShapeDtypeStruct((2, NT, 16), jnp.float32)],
    scratch_types=[pltpu.VMEM((128,), jnp.int32),
                   pltpu.VMEM((128,), jnp.int32),
                   pltpu.VMEM((128, 16), jnp.float32),
                   pltpu.VMEM((128, 16), jnp.float32),
                   pltpu.VMEM((128, 16), jnp.float32),
                   pltpu.VMEM((16,), jnp.float32),
                   pltpu.VMEM_SHARED((NT, 16), jnp.float32),
                   pltpu.SemaphoreType.DMA],
)
def _sc_pass1(*args):
    _pass1_body(*args)


def _pass2_body(src_hbm, dst_hbm, ee_hbm, rr_hbm, h_hbm, zer_hbm,
                acc_hbm,
                idxs, idxd, eebuf, rrows, webuf, hh2, msgbuf, sacc, sem0, sem1,
                *, d_width, cw, heads, l5mode):
    cid = lax.axis_index("c")
    sid = lax.axis_index("s")

    @pl.when(sid == 0)
    def _():
        pltpu.sync_copy(zer_hbm, sacc)

    plsc.subcore_barrier()
    base0 = (cid * NSUB + sid) * PER_SUB
    sub = hh2.shape[1]
    nsub = CHUNK2 // sub
    sems = [sem0, sem1]

    def chunk(k, _):
        base = base0 + k * CHUNK2
        pltpu.sync_copy(src_hbm.at[pl.ds(base, CHUNK2)], idxs)
        pltpu.sync_copy(dst_hbm.at[pl.ds(base, CHUNK2)], idxd)
        pltpu.sync_copy(ee_hbm.at[pl.ds(base, CHUNK2)], eebuf)
        pltpu.async_copy(rr_hbm.at[idxd], rrows, sem0).wait()

        @plsc.parallel_loop(0, CHUNK2, 1, unroll=8)
        def wmul(j):
            webuf[j] = eebuf[j] * rrows[j]

        descs = [pltpu.async_copy(h_hbm.at[idxs.at[pl.ds(0, sub)]], hh2.at[0], sems[0]),
                 None]
        for sj in range(nsub):
            b = sj % 2
            descs[b].wait()
            if sj + 1 < nsub:
                nb = (sj + 1) % 2
                descs[nb] = pltpu.async_copy(
                    h_hbm.at[idxs.at[pl.ds((sj + 1) * sub, sub)]], hh2.at[nb], sems[nb])

            @plsc.parallel_loop(0, sub, 1, unroll=2)
            def edge(r):
                row = sj * sub + r
                if l5mode:
                    wv = webuf[row]
                    msgbuf[row, 0:16] = wv * hh2[b, r, 0:16]
                    msgbuf[row, 16:32] = wv * hh2[b, r, 16:32]
                else:
                    wv = webuf[row]
                    gdn = lax.GatherDimensionNumbers(
                        offset_dims=(), collapsed_slice_dims=(0,),
                        start_index_map=(0,))
                    accs = [jnp.zeros((16,), jnp.float32) for _ in range(cw // 16)]
                    for h in range(heads):
                        h16 = jnp.full((16, 1), h, jnp.int32)
                        wb = lax.gather(wv, h16, gdn, (1,),
                                        mode=lax.GatherScatterMode.PROMISE_IN_BOUNDS)
                        for c in range(cw // 16):
                            accs[c] = accs[c] + wb * hh2[b, r, h * cw + c * 16: h * cw + (c + 1) * 16]
                    for c in range(cw // 16):
                        msgbuf[row, c * 16:(c + 1) * 16] = accs[c]

        pltpu.sync_copy(msgbuf, sacc.at[idxd], add=True)
        return 0

    lax.fori_loop(0, PER_SUB // CHUNK2, chunk, 0)
    plsc.subcore_barrier()

    @pl.when(sid == 0)
    def _():
        pltpu.sync_copy(sacc, acc_hbm.at[cid])


@functools.cache
def _make_pass2(d_width, cw, heads, l5mode):
    body = functools.partial(_pass2_body, d_width=d_width, cw=cw,
                             heads=heads, l5mode=l5mode)
    sub = 16

    @functools.partial(
        pl.kernel, mesh=_MESH, compiler_params=_SC_PARAMS,
        out_type=[jax.ShapeDtypeStruct((2, NT, cw), jnp.float32)],
        scratch_types=[pltpu.VMEM((CHUNK2,), jnp.int32),
                       pltpu.VMEM((CHUNK2,), jnp.int32),
                       pltpu.VMEM((CHUNK2, 16), jnp.float32),
                       pltpu.VMEM((CHUNK2, 16), jnp.float32),
                       pltpu.VMEM((CHUNK2, 16), jnp.float32),
                       pltpu.VMEM((2, sub, d_width), jnp.float32),
                       pltpu.VMEM((CHUNK2, cw), jnp.float32),
                       pltpu.VMEM_SHARED((NT, cw), jnp.float32),
                       pltpu.SemaphoreType.DMA,
                       pltpu.SemaphoreType.DMA],
    )
    def _k(*args):
        body(*args)

    return _k


def _fold_att(w, att, heads, fout):
    ws = (w.reshape(-1, heads, fout) * att[None]).sum(-1)
    return jnp.pad(ws, ((0, 0), (0, 16 - heads)))


def _gat_sc(h_in, srcp, dstp, x, w_parts, ws, wd, bpad, heads, fout, l5mode):
    asrc, adst, gmax8 = _tc_att(h_in, ws, wd)
    asrc = jnp.pad(asrc, ((0, NT - N), (0, 0)))
    adst = jnp.pad(adst, ((0, NT - N), (0, 0)))
    gmax = gmax8[0]
    zer16 = jnp.zeros((NT, 16), jnp.float32)
    ee, s2 = _sc_pass1(srcp, dstp, asrc, adst, gmax, zer16)
    s = s2[0, :N] + s2[1, :N]
    rr = 1.0 / (s + 1e-16)
    rrp = jnp.pad(rr, ((0, NT - N), (0, 0)))
    acc_parts = []
    for wp in w_parts:
        htab = _tc_proj(h_in, wp)
        d_width = wp.shape[1]
        cw = 32 if l5mode else d_width // heads
        zcw = jnp.zeros((NT, cw), jnp.float32)
        (acc2,) = _make_pass2(d_width, cw, heads, l5mode)(
            srcp, dstp, ee, rrp, htab, zcw)
        acc_parts.append(acc2[0, :N])
        acc_parts.append(acc2[1, :N])
    return acc_parts


def kernel(x, edge_index, conv_feat, Wc1, bc1, Wc2, bc2, W1, as1, ad1, b1, W2, as2, ad2, b2, W3, as3, ad3, b3, W4, as4, ad4, b4, W5, as5, ad5, b5):
    n = x.shape[0]
    dn = ('NCHW', 'OIHW', 'NCHW')
    f = jax.lax.conv_general_dilated(conv_feat, Wc1, (1, 1), 'SAME', dimension_numbers=dn) + bc1[None, :, None, None]
    f = jax.nn.relu(f)
    f = jax.lax.conv_general_dilated(f, Wc2, (1, 1), 'SAME', dimension_numbers=dn) + bc2[None, :, None, None]
    f = jax.nn.relu(f)
    f = jnp.mean(f, axis=(2, 3))
    cf = jnp.broadcast_to(f, (n, 24))
    h = jnp.concatenate([cf, x], axis=1)

    loop = jnp.arange(n, dtype=jnp.int32)
    srcp = jnp.concatenate([edge_index[0].astype(jnp.int32), loop,
                            jnp.zeros((EP - E - n,), jnp.int32)])
    dstp = jnp.concatenate([edge_index[1].astype(jnp.int32), loop,
                            jnp.full((EP - E - n,), N, jnp.int32)])

    dims = [(34, 64, 8), (64, 128, 16), (128, 256, 8), (256, 128, 8)]
    weights = [(W1, as1, ad1, b1), (W2, as2, ad2, b2), (W3, as3, ad3, b3), (W4, as4, ad4, b4)]
    for (fin, fout, heads), (W, a_s, a_d, b) in zip(dims, weights):
        ws = _fold_att(W, a_s, heads, fout)
        wd = _fold_att(W, a_d, heads, fout)
        if heads * fout > 1024:
            half = fout // 2
            wr = W.reshape(fin, heads, fout)
            w_parts = [wr[:, :, :half].reshape(fin, heads * half),
                       wr[:, :, half:].reshape(fin, heads * half)]
        else:
            w_parts = [W]
        bpad = jnp.broadcast_to(b[None, :], (8, fout))
        acc_parts = _gat_sc(h, srcp, dstp, x, w_parts, ws, wd, bpad, heads, fout, False)
        h = _tc_epi(x, bpad, acc_parts, 1.0 / heads, fout)

    ws5 = _fold_att(W5, as5, 16, 2)
    wd5 = _fold_att(W5, ad5, 16, 2)
    w5fm = W5.reshape(128, 16, 2).transpose(0, 2, 1).reshape(128, 32)
    b5pad = jnp.broadcast_to(jnp.pad(b5, (0, 14))[None, :], (8, 16))
    acc_parts = _gat_sc(h, srcp, dstp, x, [w5fm], ws5, wd5, b5pad, 16, 2, True)
    out = _tc_epi5(x, b5pad, acc_parts[0], acc_parts[1])
    return out
